# baseline (speedup 1.0000x reference)
import jax
import jax.numpy as jnp
from jax.experimental import pallas as pl

B, N, DIM = 16, 1024, 768
CB, CD = 8192, 256
TOK = B * N
TT = 256
CT = 2048
NC = CB // CT
LANES = 128
LN_EPS = 1e-5
COMMIT = 0.25

_BF = jnp.bfloat16
_NT = (((1,), (1,)), ((), ()))
_NN = (((1,), (0,)), ((), ()))


def _vq_fused_kernel(h_ref, w1_ref, b1_ref, embb_ref, esq_ref, w2_ref, b2_ref,
                     idx_ref, mind_ref, q_ref):
    flat = jax.lax.dot_general(h_ref[...], w1_ref[...],
                               _NN, preferred_element_type=jnp.float32)
    flat = flat + b1_ref[...]
    zsq = jnp.sum(flat * flat, axis=1, keepdims=True)
    flatb = flat.astype(_BF)

    lane = jax.lax.broadcasted_iota(jnp.int32, (TT, LANES), 1)

    run_val = None
    run_idx = None
    for c in range(NC):
        em = embb_ref[c * CT:(c + 1) * CT, :]
        dot = jax.lax.dot_general(flatb, em, _NT,
                                  preferred_element_type=jnp.float32)
        dist = (zsq - 2.0 * dot) + esq_ref[:, c * CT:(c + 1) * CT]
        for s in range(CT // LANES):
            vals = dist[:, s * LANES:(s + 1) * LANES]
            ids = lane + jnp.int32(c * CT + s * LANES)
            if run_val is None:
                run_val, run_idx = vals, ids
            else:
                upd = vals < run_val
                run_val = jnp.where(upd, vals, run_val)
                run_idx = jnp.where(upd, ids, run_idx)

    mind = jnp.min(run_val, axis=1, keepdims=True)
    cand = jnp.where(run_val == mind, run_idx, jnp.int32(2**31 - 1))
    ridx = jnp.min(cand, axis=1, keepdims=True)

    zq = None
    for c in range(NC):
        iota_c = jax.lax.broadcasted_iota(jnp.int32, (TT, CT), 1) \
            + jnp.int32(c * CT)
        oh = (iota_c == ridx).astype(_BF)
        em = embb_ref[c * CT:(c + 1) * CT, :]
        part = jax.lax.dot_general(oh, em, _NN,
                                   preferred_element_type=jnp.float32)
        zq = part if zq is None else zq + part

    q = jax.lax.dot_general(zq.astype(_BF), w2_ref[...],
                            _NN, preferred_element_type=jnp.float32)
    q_ref[...] = q + b2_ref[...]
    idx_ref[...] = ridx
    mind_ref[...] = mind


def _esq_kernel(embed_ref, esq_ref):
    em = embed_ref[...]
    esq_ref[...] = jnp.sum(em * em, axis=1, keepdims=True)


_CPT = 1024


def kernel(x, ln_gamma, ln_beta, W1, b1, embed, W2, b2):
    xf = x.reshape(TOK, DIM).astype(jnp.float32)
    mu = jnp.mean(xf, axis=-1, keepdims=True)
    var = jnp.mean((xf - mu) ** 2, axis=-1, keepdims=True)
    x_normed = (xf - mu) / jnp.sqrt(var + LN_EPS) * ln_gamma + ln_beta
    h = jax.nn.gelu(x_normed, approximate=False).astype(_BF)
    embb = embed.astype(_BF)
    w1b = W1.astype(_BF)
    w2b = W2.astype(_BF)

    esq2 = pl.pallas_call(
        _esq_kernel,
        grid=(CB // _CPT,),
        in_specs=[pl.BlockSpec((_CPT, CD), lambda i: (i, 0))],
        out_specs=pl.BlockSpec((_CPT, 1), lambda i: (i, 0)),
        out_shape=jax.ShapeDtypeStruct((CB, 1), jnp.float32),
    )(embed)
    esq_row = esq2.reshape(1, CB)

    idx, mind, q = pl.pallas_call(
        _vq_fused_kernel,
        grid=(TOK // TT,),
        in_specs=[
            pl.BlockSpec((TT, DIM), lambda i: (i, 0)),
            pl.BlockSpec((DIM, CD), lambda i: (0, 0)),
            pl.BlockSpec((CD,), lambda i: (0,)),
            pl.BlockSpec((CB, CD), lambda i: (0, 0)),
            pl.BlockSpec((1, CB), lambda i: (0, 0)),
            pl.BlockSpec((CD, DIM), lambda i: (0, 0)),
            pl.BlockSpec((DIM,), lambda i: (0,)),
        ],
        out_specs=[
            pl.BlockSpec((TT, 1), lambda i: (i, 0)),
            pl.BlockSpec((TT, 1), lambda i: (i, 0)),
            pl.BlockSpec((TT, DIM), lambda i: (i, 0)),
        ],
        out_shape=[
            jax.ShapeDtypeStruct((TOK, 1), jnp.int32),
            jax.ShapeDtypeStruct((TOK, 1), jnp.float32),
            jax.ShapeDtypeStruct((TOK, DIM), jnp.float32),
        ],
    )(h, w1b, b1, embb, esq_row, w2b, b2)

    indices = idx.reshape(B, N)
    quantized = q.reshape(B, N, DIM)
    commitment_loss = COMMIT * (jnp.sum(mind) / (TOK * CD))
    return quantized, indices, commitment_loss


# speedup vs baseline: 1.0007x; 1.0007x over previous
import jax
import jax.numpy as jnp
from jax.experimental import pallas as pl

B, N, DIM = 16, 1024, 768
CB, CD = 8192, 256
TOK = B * N
TT = 256
CT = 2048
NC = CB // CT
LANES = 128
LN_EPS = 1e-5
COMMIT = 0.25

_BF = jnp.bfloat16
_NT = (((1,), (1,)), ((), ()))
_NN = (((1,), (0,)), ((), ()))


def _vq_fused_kernel(h_ref, w1_ref, b1_ref, embb_ref, embf_ref, esq_ref,
                     w2_ref, b2_ref, idx_ref, mind_ref, q_ref):
    flat = jax.lax.dot_general(h_ref[...], w1_ref[...],
                               _NN, preferred_element_type=jnp.float32)
    flat = flat + b1_ref[...]
    zsq = jnp.sum(flat * flat, axis=1, keepdims=True)
    nflatb = (-2.0 * flat).astype(_BF)

    lane = jax.lax.broadcasted_iota(jnp.int32, (TT, LANES), 1)

    run_val = None
    run_idx = None
    for c in range(NC):
        em = embb_ref[c * CT:(c + 1) * CT, :]
        ndot = jax.lax.dot_general(nflatb, em, _NT,
                                   preferred_element_type=jnp.float32)
        dist = (zsq + ndot) + esq_ref[:, c * CT:(c + 1) * CT]
        for s in range(CT // LANES):
            vals = dist[:, s * LANES:(s + 1) * LANES]
            ids = lane + jnp.int32(c * CT + s * LANES)
            if run_val is None:
                run_val, run_idx = vals, ids
            else:
                upd = vals < run_val
                run_val = jnp.where(upd, vals, run_val)
                run_idx = jnp.where(upd, ids, run_idx)

    mind = jnp.min(run_val, axis=1, keepdims=True)
    cand = jnp.where(run_val == mind, run_idx, jnp.int32(2**31 - 1))
    ridx = jnp.min(cand, axis=1, keepdims=True)

    iota = jax.lax.broadcasted_iota(jnp.int32, (TT, CT), 1)
    fone = jnp.ones((), jnp.float32)
    fzero = jnp.zeros((), jnp.float32)
    zq = None
    for c in range(NC):
        oh = jnp.where(iota == ridx - jnp.int32(c * CT), fone, fzero)
        em = embf_ref[c * CT:(c + 1) * CT, :]
        part = jax.lax.dot_general(oh, em, _NN,
                                   preferred_element_type=jnp.float32)
        zq = part if zq is None else zq + part

    q = jax.lax.dot_general(zq.astype(_BF), w2_ref[...],
                            _NN, preferred_element_type=jnp.float32)
    q_ref[...] = q + b2_ref[...]
    idx_ref[...] = ridx
    mind_ref[...] = mind


def _esq_kernel(embed_ref, esq_ref):
    em = embed_ref[...]
    esq_ref[...] = jnp.sum(em * em, axis=1, keepdims=True)


_CPT = 1024


def kernel(x, ln_gamma, ln_beta, W1, b1, embed, W2, b2):
    xf = x.reshape(TOK, DIM).astype(jnp.float32)
    mu = jnp.mean(xf, axis=-1, keepdims=True)
    var = jnp.mean((xf - mu) ** 2, axis=-1, keepdims=True)
    x_normed = (xf - mu) / jnp.sqrt(var + LN_EPS) * ln_gamma + ln_beta
    h = jax.nn.gelu(x_normed, approximate=False).astype(_BF)
    embb = embed.astype(_BF)
    w1b = W1.astype(_BF)
    w2b = W2.astype(_BF)

    esq2 = pl.pallas_call(
        _esq_kernel,
        grid=(CB // _CPT,),
        in_specs=[pl.BlockSpec((_CPT, CD), lambda i: (i, 0))],
        out_specs=pl.BlockSpec((_CPT, 1), lambda i: (i, 0)),
        out_shape=jax.ShapeDtypeStruct((CB, 1), jnp.float32),
    )(embed)
    esq_row = esq2.reshape(1, CB)

    idx, mind, q = pl.pallas_call(
        _vq_fused_kernel,
        grid=(TOK // TT,),
        in_specs=[
            pl.BlockSpec((TT, DIM), lambda i: (i, 0)),
            pl.BlockSpec((DIM, CD), lambda i: (0, 0)),
            pl.BlockSpec((CD,), lambda i: (0,)),
            pl.BlockSpec((CB, CD), lambda i: (0, 0)),
            pl.BlockSpec((CB, CD), lambda i: (0, 0)),
            pl.BlockSpec((1, CB), lambda i: (0, 0)),
            pl.BlockSpec((CD, DIM), lambda i: (0, 0)),
            pl.BlockSpec((DIM,), lambda i: (0,)),
        ],
        out_specs=[
            pl.BlockSpec((TT, 1), lambda i: (i, 0)),
            pl.BlockSpec((TT, 1), lambda i: (i, 0)),
            pl.BlockSpec((TT, DIM), lambda i: (i, 0)),
        ],
        out_shape=[
            jax.ShapeDtypeStruct((TOK, 1), jnp.int32),
            jax.ShapeDtypeStruct((TOK, 1), jnp.float32),
            jax.ShapeDtypeStruct((TOK, DIM), jnp.float32),
        ],
    )(h, w1b, b1, embb, embed, esq_row, w2b, b2)

    indices = idx.reshape(B, N)
    quantized = q.reshape(B, N, DIM)
    commitment_loss = COMMIT * (jnp.sum(mind) / (TOK * CD))
    return quantized, indices, commitment_loss


# speedup vs baseline: 1.0487x; 1.0479x over previous
import jax
import jax.numpy as jnp
from jax.experimental import pallas as pl

B, N, DIM = 16, 1024, 768
CB, CD = 8192, 256
TOK = B * N
TT = 512
CT = 1024
NC = CB // CT
LANES = 128
LN_EPS = 1e-5
COMMIT = 0.25

_BF = jnp.bfloat16
_NT = (((1,), (1,)), ((), ()))
_NN = (((1,), (0,)), ((), ()))


def _vq_fused_kernel(h_ref, w1_ref, b1_ref, embb_ref, embf_ref, esq_ref,
                     w2_ref, b2_ref, idx_ref, mind_ref, q_ref):
    flat = jax.lax.dot_general(h_ref[...], w1_ref[...],
                               _NN, preferred_element_type=jnp.float32)
    flat = flat + b1_ref[...]
    zsq = jnp.sum(flat * flat, axis=1, keepdims=True)
    nflatb = (-2.0 * flat).astype(_BF)

    lane = jax.lax.broadcasted_iota(jnp.int32, (TT, LANES), 1)

    run_val = None
    run_idx = None
    for c in range(NC):
        em = embb_ref[c * CT:(c + 1) * CT, :]
        ndot = jax.lax.dot_general(nflatb, em, _NT,
                                   preferred_element_type=jnp.float32)
        dist = (zsq + ndot) + esq_ref[:, c * CT:(c + 1) * CT]
        for s in range(CT // LANES):
            vals = dist[:, s * LANES:(s + 1) * LANES]
            ids = lane + jnp.int32(c * CT + s * LANES)
            if run_val is None:
                run_val, run_idx = vals, ids
            else:
                upd = vals < run_val
                run_val = jnp.where(upd, vals, run_val)
                run_idx = jnp.where(upd, ids, run_idx)

    mind = jnp.min(run_val, axis=1, keepdims=True)
    cand = jnp.where(run_val == mind, run_idx, jnp.int32(2**31 - 1))
    ridx = jnp.min(cand, axis=1, keepdims=True)

    iota = jax.lax.broadcasted_iota(jnp.int32, (TT, CT), 1)
    fone = jnp.ones((), jnp.float32)
    fzero = jnp.zeros((), jnp.float32)
    zq = None
    for c in range(NC):
        oh = jnp.where(iota == ridx - jnp.int32(c * CT), fone, fzero)
        em = embf_ref[c * CT:(c + 1) * CT, :]
        part = jax.lax.dot_general(oh, em, _NN,
                                   preferred_element_type=jnp.float32)
        zq = part if zq is None else zq + part

    q = jax.lax.dot_general(zq.astype(_BF), w2_ref[...],
                            _NN, preferred_element_type=jnp.float32)
    q_ref[...] = q + b2_ref[...]
    idx_ref[...] = ridx
    mind_ref[...] = mind


def _esq_kernel(embed_ref, esq_ref):
    em = embed_ref[...]
    esq_ref[...] = jnp.sum(em * em, axis=1, keepdims=True)


_CPT = 1024


def kernel(x, ln_gamma, ln_beta, W1, b1, embed, W2, b2):
    xf = x.reshape(TOK, DIM).astype(jnp.float32)
    mu = jnp.mean(xf, axis=-1, keepdims=True)
    var = jnp.mean((xf - mu) ** 2, axis=-1, keepdims=True)
    x_normed = (xf - mu) / jnp.sqrt(var + LN_EPS) * ln_gamma + ln_beta
    h = jax.nn.gelu(x_normed, approximate=False).astype(_BF)
    embb = embed.astype(_BF)
    w1b = W1.astype(_BF)
    w2b = W2.astype(_BF)

    esq2 = pl.pallas_call(
        _esq_kernel,
        grid=(CB // _CPT,),
        in_specs=[pl.BlockSpec((_CPT, CD), lambda i: (i, 0))],
        out_specs=pl.BlockSpec((_CPT, 1), lambda i: (i, 0)),
        out_shape=jax.ShapeDtypeStruct((CB, 1), jnp.float32),
    )(embed)
    esq_row = esq2.reshape(1, CB)

    idx, mind, q = pl.pallas_call(
        _vq_fused_kernel,
        grid=(TOK // TT,),
        in_specs=[
            pl.BlockSpec((TT, DIM), lambda i: (i, 0)),
            pl.BlockSpec((DIM, CD), lambda i: (0, 0)),
            pl.BlockSpec((CD,), lambda i: (0,)),
            pl.BlockSpec((CB, CD), lambda i: (0, 0)),
            pl.BlockSpec((CB, CD), lambda i: (0, 0)),
            pl.BlockSpec((1, CB), lambda i: (0, 0)),
            pl.BlockSpec((CD, DIM), lambda i: (0, 0)),
            pl.BlockSpec((DIM,), lambda i: (0,)),
        ],
        out_specs=[
            pl.BlockSpec((TT, 1), lambda i: (i, 0)),
            pl.BlockSpec((TT, 1), lambda i: (i, 0)),
            pl.BlockSpec((TT, DIM), lambda i: (i, 0)),
        ],
        out_shape=[
            jax.ShapeDtypeStruct((TOK, 1), jnp.int32),
            jax.ShapeDtypeStruct((TOK, 1), jnp.float32),
            jax.ShapeDtypeStruct((TOK, DIM), jnp.float32),
        ],
    )(h, w1b, b1, embb, embed, esq_row, w2b, b2)

    indices = idx.reshape(B, N)
    quantized = q.reshape(B, N, DIM)
    commitment_loss = COMMIT * (jnp.sum(mind) / (TOK * CD))
    return quantized, indices, commitment_loss


# speedup vs baseline: 1.0834x; 1.0331x over previous
import jax
import jax.numpy as jnp
from jax.experimental import pallas as pl

B, N, DIM = 16, 1024, 768
CB, CD = 8192, 256
TOK = B * N
TT = 512
CT = 1024
NC = CB // CT
LANES = 128
LN_EPS = 1e-5
COMMIT = 0.25

_BF = jnp.bfloat16
_NT = (((1,), (1,)), ((), ()))
_NN = (((1,), (0,)), ((), ()))


def _vq_fused_kernel(nflat_ref, zsq_ref, embb_ref, embf_ref, esq_ref,
                     w2_ref, b2_ref, idx_ref, mind_ref, q_ref):
    nflatb = nflat_ref[...]
    zsq = zsq_ref[...]

    lane = jax.lax.broadcasted_iota(jnp.int32, (TT, LANES), 1)

    run_val = None
    run_idx = None
    for c in range(NC):
        em = embb_ref[c * CT:(c + 1) * CT, :]
        ndot = jax.lax.dot_general(nflatb, em, _NT,
                                   preferred_element_type=jnp.float32)
        dist = (zsq + ndot) + esq_ref[:, c * CT:(c + 1) * CT]
        for s in range(CT // LANES):
            vals = dist[:, s * LANES:(s + 1) * LANES]
            ids = lane + jnp.int32(c * CT + s * LANES)
            if run_val is None:
                run_val, run_idx = vals, ids
            else:
                upd = vals < run_val
                run_val = jnp.where(upd, vals, run_val)
                run_idx = jnp.where(upd, ids, run_idx)

    mind = jnp.min(run_val, axis=1, keepdims=True)
    cand = jnp.where(run_val == mind, run_idx, jnp.int32(2**31 - 1))
    ridx = jnp.min(cand, axis=1, keepdims=True)

    iota = jax.lax.broadcasted_iota(jnp.int32, (TT, CT), 1)
    fone = jnp.ones((), jnp.float32)
    fzero = jnp.zeros((), jnp.float32)
    zq = None
    for c in range(NC):
        oh = jnp.where(iota == ridx - jnp.int32(c * CT), fone, fzero)
        em = embf_ref[c * CT:(c + 1) * CT, :]
        part = jax.lax.dot_general(oh, em, _NN,
                                   preferred_element_type=jnp.float32)
        zq = part if zq is None else zq + part

    q = jax.lax.dot_general(zq.astype(_BF), w2_ref[...],
                            _NN, preferred_element_type=jnp.float32)
    q_ref[...] = q + b2_ref[...]
    idx_ref[...] = ridx
    mind_ref[...] = mind


def kernel(x, ln_gamma, ln_beta, W1, b1, embed, W2, b2):
    xf = x.reshape(TOK, DIM).astype(jnp.float32)
    mu = jnp.mean(xf, axis=-1, keepdims=True)
    var = jnp.mean((xf - mu) ** 2, axis=-1, keepdims=True)
    x_normed = (xf - mu) / jnp.sqrt(var + LN_EPS) * ln_gamma + ln_beta
    h = jax.nn.gelu(x_normed, approximate=False)
    flat = (h @ W1 + b1).reshape(TOK, CD)
    zsq = jnp.sum(flat ** 2, axis=1, keepdims=True)
    nflatb = (-2.0 * flat).astype(_BF)
    esq_row = jnp.sum(embed ** 2, axis=1, keepdims=True).T
    embb = embed.astype(_BF)
    w2b = W2.astype(_BF)

    idx, mind, q = pl.pallas_call(
        _vq_fused_kernel,
        grid=(TOK // TT,),
        in_specs=[
            pl.BlockSpec((TT, CD), lambda i: (i, 0)),
            pl.BlockSpec((TT, 1), lambda i: (i, 0)),
            pl.BlockSpec((CB, CD), lambda i: (0, 0)),
            pl.BlockSpec((CB, CD), lambda i: (0, 0)),
            pl.BlockSpec((1, CB), lambda i: (0, 0)),
            pl.BlockSpec((CD, DIM), lambda i: (0, 0)),
            pl.BlockSpec((DIM,), lambda i: (0,)),
        ],
        out_specs=[
            pl.BlockSpec((TT, 1), lambda i: (i, 0)),
            pl.BlockSpec((TT, 1), lambda i: (i, 0)),
            pl.BlockSpec((TT, DIM), lambda i: (i, 0)),
        ],
        out_shape=[
            jax.ShapeDtypeStruct((TOK, 1), jnp.int32),
            jax.ShapeDtypeStruct((TOK, 1), jnp.float32),
            jax.ShapeDtypeStruct((TOK, DIM), jnp.float32),
        ],
    )(nflatb, zsq, embb, embed, esq_row, w2b, b2)

    indices = idx.reshape(B, N)
    quantized = q.reshape(B, N, DIM)
    commitment_loss = COMMIT * (jnp.sum(mind) / (TOK * CD))
    return quantized, indices, commitment_loss
